# baseline (device time: 83576 ns/iter reference)
import jax
import jax.numpy as jnp
from jax import lax
from jax.experimental import pallas as pl
from jax.experimental.pallas import tpu as pltpu

N_DEV = 4
SQ = 1024
SKV = 1024
HQ_LOCAL = 8
DH = 128
D_LOCAL = HQ_LOCAL * DH
D_MODEL = 1024
BLK = 64
SCALE = 0.08838834764831843
NEG = -1e9

RC = SQ // N_DEV


def kernel(x, Wq, K_ext, V_ext, Wo):
    x2 = x[0]
    k2 = K_ext[0]
    v2 = V_ext[0]

    def body(x_ref, wq_ref, k_ref, v_ref, wo_ref, out_ref,
             ctx_ref, ps_ref, rs_recv_ref, ag_src_ref, ag_recv_ref,
             wq_l_ref, wo_l_ref,
             dma_sems, rs_send_sems, rs_recv_sems, ag_send_sems,
             ag_recv_sems):
        my = lax.axis_index("i")

        def cmod(k):
            return lax.rem(k + 4 * N_DEV, N_DEV)

        cp_wq = pltpu.make_async_copy(
            wq_ref.at[:, pl.ds(my * D_LOCAL, D_LOCAL)], wq_l_ref,
            dma_sems.at[0])
        cp_wo = pltpu.make_async_copy(
            wo_ref.at[pl.ds(my * D_LOCAL, D_LOCAL), :], wo_l_ref,
            dma_sems.at[1])
        cp_wq.start()
        cp_wo.start()

        barrier_sem = pltpu.get_barrier_semaphore()
        for d in range(1, N_DEV):
            pl.semaphore_signal(
                barrier_sem, inc=1,
                device_id=(cmod(my + d),),
                device_id_type=pl.DeviceIdType.MESH,
            )
        pl.semaphore_wait(barrier_sem, N_DEV - 1)

        cp_wq.wait()
        cp_wo.wait()
        wq_l = (wq_l_ref[...] * SCALE).astype(jnp.bfloat16)
        wo_l = wo_l_ref[...].astype(jnp.bfloat16)
        xb = x_ref[...].astype(jnp.bfloat16)

        di = lax.broadcasted_iota(jnp.int32, (RC, RC), 0) // BLK
        dj = lax.broadcasted_iota(jnp.int32, (RC, RC), 1) // BLK
        diag_mask = dj <= di

        rs_sends = []
        for c in range(N_DEV):
            KL = RC * (c + 1)
            q_c = jnp.dot(xb[c * RC:(c + 1) * RC, :], wq_l,
                          preferred_element_type=jnp.float32
                          ).astype(jnp.bfloat16)
            for h in range(HQ_LOCAL):
                sl = slice(h * DH, (h + 1) * DH)
                s = lax.dot_general(
                    q_c[:, sl], k_ref[:KL, h, :].astype(jnp.bfloat16),
                    (((1,), (1,)), ((), ())),
                    preferred_element_type=jnp.float32,
                )
                if c == 0:
                    s = jnp.where(diag_mask, s, NEG)
                else:
                    s = jnp.concatenate(
                        [s[:, :c * RC],
                         jnp.where(diag_mask, s[:, c * RC:], NEG)], axis=1)
                m = jnp.max(s, axis=1, keepdims=True)
                p = jnp.exp(s - m)
                denom = jnp.sum(p, axis=1, keepdims=True)
                ctx = jnp.dot(p.astype(jnp.bfloat16),
                              v_ref[:KL, h, :].astype(jnp.bfloat16),
                              preferred_element_type=jnp.float32)
                ctx_ref[c * RC:(c + 1) * RC, sl] = (
                    ctx / denom).astype(jnp.bfloat16)
            pc = jnp.dot(ctx_ref[c * RC:(c + 1) * RC, :], wo_l,
                         preferred_element_type=jnp.float32)
            ps_ref[c] = pc.astype(jnp.bfloat16)

            rdma = pltpu.make_async_remote_copy(
                src_ref=ps_ref.at[c],
                dst_ref=rs_recv_ref.at[cmod(my - c - 1)],
                send_sem=rs_send_sems.at[c],
                recv_sem=rs_recv_sems.at[cmod(my - c - 1)],
                device_id=(c,),
                device_id_type=pl.DeviceIdType.MESH,
            )
            rdma.start()
            rs_sends.append(rdma)

        for j in range(N_DEV):
            pltpu.make_async_remote_copy(
                src_ref=rs_recv_ref.at[j], dst_ref=rs_recv_ref.at[j],
                send_sem=rs_send_sems.at[0], recv_sem=rs_recv_sems.at[j],
                device_id=(my,), device_id_type=pl.DeviceIdType.MESH,
            ).wait_recv()

        red = ((rs_recv_ref[0].astype(jnp.float32)
                + rs_recv_ref[1].astype(jnp.float32))
               + (rs_recv_ref[2].astype(jnp.float32)
                  + rs_recv_ref[3].astype(jnp.float32)))
        out_ref[pl.ds(my * RC, RC), :] = red
        ag_src_ref[...] = red.astype(jnp.bfloat16)

        ag_sends = []
        for d in range(1, N_DEV):
            rdma = pltpu.make_async_remote_copy(
                src_ref=ag_src_ref,
                dst_ref=ag_recv_ref.at[3 - d],
                send_sem=ag_send_sems.at[d - 1],
                recv_sem=ag_recv_sems.at[3 - d],
                device_id=(cmod(my + d),),
                device_id_type=pl.DeviceIdType.MESH,
            )
            rdma.start()
            ag_sends.append(rdma)

        for j in range(N_DEV - 1):
            pltpu.make_async_remote_copy(
                src_ref=ag_src_ref, dst_ref=ag_recv_ref.at[j],
                send_sem=ag_send_sems.at[0], recv_sem=ag_recv_sems.at[j],
                device_id=(my,), device_id_type=pl.DeviceIdType.MESH,
            ).wait_recv()
            src_chip = cmod(my + j + 1)
            out_ref[pl.ds(src_chip * RC, RC), :] = (
                ag_recv_ref[j].astype(jnp.float32))

        for rdma in rs_sends:
            rdma.wait_send()
        for rdma in ag_sends:
            rdma.wait_send()

    out = pl.pallas_call(
        body,
        out_shape=jax.ShapeDtypeStruct((SQ, D_MODEL), jnp.float32),
        in_specs=[
            pl.BlockSpec(memory_space=pltpu.MemorySpace.VMEM),
            pl.BlockSpec(memory_space=pl.ANY),
            pl.BlockSpec(memory_space=pltpu.MemorySpace.VMEM),
            pl.BlockSpec(memory_space=pltpu.MemorySpace.VMEM),
            pl.BlockSpec(memory_space=pl.ANY),
        ],
        out_specs=pl.BlockSpec(memory_space=pltpu.MemorySpace.VMEM),
        scratch_shapes=[
            pltpu.VMEM((SQ, D_LOCAL), jnp.bfloat16),
            pltpu.VMEM((N_DEV, RC, D_MODEL), jnp.bfloat16),
            pltpu.VMEM((N_DEV, RC, D_MODEL), jnp.bfloat16),
            pltpu.VMEM((RC, D_MODEL), jnp.bfloat16),
            pltpu.VMEM((N_DEV - 1, RC, D_MODEL), jnp.bfloat16),
            pltpu.VMEM((D_MODEL, D_LOCAL), jnp.float32),
            pltpu.VMEM((D_LOCAL, D_MODEL), jnp.float32),
            pltpu.SemaphoreType.DMA((2,)),
            pltpu.SemaphoreType.DMA((N_DEV,)),
            pltpu.SemaphoreType.DMA((N_DEV,)),
            pltpu.SemaphoreType.DMA((N_DEV - 1,)),
            pltpu.SemaphoreType.DMA((N_DEV - 1,)),
        ],
        compiler_params=pltpu.CompilerParams(
            collective_id=0,
            vmem_limit_bytes=100 * 1024 * 1024,
        ),
    )(x2, Wq, k2, v2, Wo)
    return out[None]


# device time: 68249 ns/iter; 1.2246x vs baseline; 1.2246x over previous
import jax
import jax.numpy as jnp
from jax import lax
from jax.experimental import pallas as pl
from jax.experimental.pallas import tpu as pltpu

N_DEV = 4
SQ = 1024
SKV = 1024
HQ_LOCAL = 8
DH = 128
D_LOCAL = HQ_LOCAL * DH
D_MODEL = 1024
BLK = 64
SCALE = 0.08838834764831843
NEG = -1e9

RC = SQ // N_DEV


def kernel(x, Wq, K_ext, V_ext, Wo):
    x2 = x[0]
    k2 = K_ext.reshape(SKV, HQ_LOCAL * DH).astype(jnp.bfloat16)
    v2 = V_ext.reshape(SKV, HQ_LOCAL * DH).astype(jnp.bfloat16)

    def body(x_ref, wq_ref, k_ref, v_ref, wo_ref, out_ref,
             ctx_ref, ps_ref, rs_recv_ref, ag_src_ref, ag_recv_ref,
             wq_l_ref, wo_l_ref,
             dma_sems, rs_send_sems, rs_recv_sems, ag_send_sems,
             ag_recv_sems):
        my = lax.axis_index("i")

        def cmod(k):
            return lax.rem(k + 4 * N_DEV, N_DEV)

        cp_wq = pltpu.make_async_copy(
            wq_ref.at[:, pl.ds(my * D_LOCAL, D_LOCAL)], wq_l_ref,
            dma_sems.at[0])
        cp_wo = pltpu.make_async_copy(
            wo_ref.at[pl.ds(my * D_LOCAL, D_LOCAL), :], wo_l_ref,
            dma_sems.at[1])
        cp_wq.start()
        cp_wo.start()

        barrier_sem = pltpu.get_barrier_semaphore()
        for d in range(1, N_DEV):
            pl.semaphore_signal(
                barrier_sem, inc=1,
                device_id=(cmod(my + d),),
                device_id_type=pl.DeviceIdType.MESH,
            )
        pl.semaphore_wait(barrier_sem, N_DEV - 1)

        cp_wq.wait()
        cp_wo.wait()
        wq_l = (wq_l_ref[...] * SCALE).astype(jnp.bfloat16)
        wo_l = wo_l_ref[...].astype(jnp.bfloat16)
        xb = x_ref[...].astype(jnp.bfloat16)

        di = lax.broadcasted_iota(jnp.int32, (RC, RC), 0) // BLK
        dj = lax.broadcasted_iota(jnp.int32, (RC, RC), 1) // BLK
        diag_mask = dj <= di

        rs_sends = []
        for c in range(N_DEV):
            KL = RC * (c + 1)
            q_c = jnp.dot(xb[c * RC:(c + 1) * RC, :], wq_l,
                          preferred_element_type=jnp.float32
                          ).astype(jnp.bfloat16)
            for h in range(HQ_LOCAL):
                sl = slice(h * DH, (h + 1) * DH)
                s = lax.dot_general(
                    q_c[:, sl], k_ref[:KL, sl], (((1,), (1,)), ((), ())),
                    preferred_element_type=jnp.float32,
                )
                if c == 0:
                    s = jnp.where(diag_mask, s, NEG)
                else:
                    s = jnp.concatenate(
                        [s[:, :c * RC],
                         jnp.where(diag_mask, s[:, c * RC:], NEG)], axis=1)
                m = jnp.max(s, axis=1, keepdims=True)
                p = jnp.exp(s - m)
                denom = jnp.sum(p, axis=1, keepdims=True)
                ctx = jnp.dot(p.astype(jnp.bfloat16), v_ref[:KL, sl],
                              preferred_element_type=jnp.float32)
                ctx_ref[c * RC:(c + 1) * RC, sl] = (
                    ctx / denom).astype(jnp.bfloat16)
            pc = jnp.dot(ctx_ref[c * RC:(c + 1) * RC, :], wo_l,
                         preferred_element_type=jnp.float32)
            ps_ref[c] = pc.astype(jnp.bfloat16)

            rdma = pltpu.make_async_remote_copy(
                src_ref=ps_ref.at[c],
                dst_ref=rs_recv_ref.at[cmod(my - c - 1)],
                send_sem=rs_send_sems.at[c],
                recv_sem=rs_recv_sems.at[cmod(my - c - 1)],
                device_id=(c,),
                device_id_type=pl.DeviceIdType.MESH,
            )
            rdma.start()
            rs_sends.append(rdma)

        for j in range(N_DEV):
            pltpu.make_async_remote_copy(
                src_ref=rs_recv_ref.at[j], dst_ref=rs_recv_ref.at[j],
                send_sem=rs_send_sems.at[0], recv_sem=rs_recv_sems.at[j],
                device_id=(my,), device_id_type=pl.DeviceIdType.MESH,
            ).wait_recv()

        red = ((rs_recv_ref[0].astype(jnp.float32)
                + rs_recv_ref[1].astype(jnp.float32))
               + (rs_recv_ref[2].astype(jnp.float32)
                  + rs_recv_ref[3].astype(jnp.float32)))
        out_ref[pl.ds(my * RC, RC), :] = red
        ag_src_ref[...] = red.astype(jnp.bfloat16)

        ag_sends = []
        for d in range(1, N_DEV):
            rdma = pltpu.make_async_remote_copy(
                src_ref=ag_src_ref,
                dst_ref=ag_recv_ref.at[3 - d],
                send_sem=ag_send_sems.at[d - 1],
                recv_sem=ag_recv_sems.at[3 - d],
                device_id=(cmod(my + d),),
                device_id_type=pl.DeviceIdType.MESH,
            )
            rdma.start()
            ag_sends.append(rdma)

        for j in range(N_DEV - 1):
            pltpu.make_async_remote_copy(
                src_ref=ag_src_ref, dst_ref=ag_recv_ref.at[j],
                send_sem=ag_send_sems.at[0], recv_sem=ag_recv_sems.at[j],
                device_id=(my,), device_id_type=pl.DeviceIdType.MESH,
            ).wait_recv()
            src_chip = cmod(my + j + 1)
            out_ref[pl.ds(src_chip * RC, RC), :] = (
                ag_recv_ref[j].astype(jnp.float32))

        for rdma in rs_sends:
            rdma.wait_send()
        for rdma in ag_sends:
            rdma.wait_send()

    out = pl.pallas_call(
        body,
        out_shape=jax.ShapeDtypeStruct((SQ, D_MODEL), jnp.float32),
        in_specs=[
            pl.BlockSpec(memory_space=pltpu.MemorySpace.VMEM),
            pl.BlockSpec(memory_space=pl.ANY),
            pl.BlockSpec(memory_space=pltpu.MemorySpace.VMEM),
            pl.BlockSpec(memory_space=pltpu.MemorySpace.VMEM),
            pl.BlockSpec(memory_space=pl.ANY),
        ],
        out_specs=pl.BlockSpec(memory_space=pltpu.MemorySpace.VMEM),
        scratch_shapes=[
            pltpu.VMEM((SQ, D_LOCAL), jnp.bfloat16),
            pltpu.VMEM((N_DEV, RC, D_MODEL), jnp.bfloat16),
            pltpu.VMEM((N_DEV, RC, D_MODEL), jnp.bfloat16),
            pltpu.VMEM((RC, D_MODEL), jnp.bfloat16),
            pltpu.VMEM((N_DEV - 1, RC, D_MODEL), jnp.bfloat16),
            pltpu.VMEM((D_MODEL, D_LOCAL), jnp.float32),
            pltpu.VMEM((D_LOCAL, D_MODEL), jnp.float32),
            pltpu.SemaphoreType.DMA((2,)),
            pltpu.SemaphoreType.DMA((N_DEV,)),
            pltpu.SemaphoreType.DMA((N_DEV,)),
            pltpu.SemaphoreType.DMA((N_DEV - 1,)),
            pltpu.SemaphoreType.DMA((N_DEV - 1,)),
        ],
        compiler_params=pltpu.CompilerParams(
            collective_id=0,
            vmem_limit_bytes=100 * 1024 * 1024,
        ),
    )(x2, Wq, k2, v2, Wo)
    return out[None]
